# baseline (speedup 1.0000x reference)
import functools

import jax
import jax.numpy as jnp
from jax import lax
from jax.experimental import pallas as pl
from jax.experimental.pallas import tpu as pltpu
from jax.experimental.pallas import tpu_sc as plsc

N_USERS = 10000
N_ITEMS = 40000
N_NODES = N_USERS + N_ITEMS
EMB = 64
HALF = EMB // 2
N_EDGES = 800000
NC = 2
NS = 16
LANES = 16

C = 128
K = 20
E_PAD = 819200
CPT = E_PAD // NS // C
NSUP = CPT // K
RB = 200
NRBLK = N_NODES // RB


def _layer_body(col_hbm, row_hbm, val_hbm, x_hbm, out_hbm,
                acc, colbuf, rowbuf, valbuf, gb0, gb1, sb0, sb1, zbuf,
                sem0, sem1, ssem0, ssem1):
    cid = lax.axis_index("c")
    sid = lax.axis_index("s")

    def zrow(i, _):
        zbuf[i, pl.ds(0, LANES)] = jnp.zeros((LANES,), jnp.float32)
        zbuf[i, pl.ds(LANES, LANES)] = jnp.zeros((LANES,), jnp.float32)
        return 0
    lax.fori_loop(0, RB, zrow, 0)

    def blk_off(i):
        return pl.multiple_of((i * NS + sid) * RB, RB)

    for i in range(-(-NRBLK // NS)):
        @pl.when(i * NS + sid < NRBLK)
        def _():
            pltpu.sync_copy(zbuf, acc.at[pl.ds(blk_off(i), RB)])
    plsc.subcore_barrier()

    tile_chunk0 = sid * CPT

    def issue(j, gb, sem):
        pltpu.async_copy(x_hbm.at[cid].at[colbuf.at[j]], gb, sem)

    def wait_gather(j, gb, sem):
        pltpu.make_async_copy(x_hbm.at[cid].at[colbuf.at[j]], gb, sem).wait()

    def scale(j, gb, sb):
        def group(g, _):
            vvec = valbuf[j, pl.ds(LANES * g, LANES)]
            for ei in range(LANES):
                e = LANES * g + ei
                v = vvec[ei]
                sb[e, pl.ds(0, LANES)] = gb[e, pl.ds(0, LANES)] * v
                sb[e, pl.ds(LANES, LANES)] = gb[e, pl.ds(LANES, LANES)] * v
            return 0
        lax.fori_loop(0, C // LANES, group, 0)

    def issue_scatter(j, sb, sem):
        pltpu.async_copy(sb, acc.at[rowbuf.at[j]], sem, add=True)

    def wait_scatter(j, sb, sem):
        pltpu.make_async_copy(sb, acc.at[rowbuf.at[j]], sem).wait()

    for s in range(NSUP):
        sup0 = pl.multiple_of(tile_chunk0 + s * K, K)
        pltpu.sync_copy(col_hbm.at[pl.ds(sup0, K)], colbuf)
        pltpu.sync_copy(row_hbm.at[pl.ds(sup0, K)], rowbuf)
        pltpu.sync_copy(val_hbm.at[pl.ds(sup0, K)], valbuf)

        issue(0, gb0, sem0)

        def pair(j2, _):
            j = 2 * j2
            issue(j + 1, gb1, sem1)
            wait_gather(j, gb0, sem0)

            @pl.when(j2 > 0)
            def _():
                wait_scatter(j, sb0, ssem0)
            scale(j, gb0, sb0)

            @pl.when(j2 < K // 2 - 1)
            def _():
                issue(j + 2, gb0, sem0)
            issue_scatter(j, sb0, ssem0)

            wait_gather(j + 1, gb1, sem1)

            @pl.when(j2 > 0)
            def _():
                wait_scatter(j, sb1, ssem1)
            scale(j + 1, gb1, sb1)
            issue_scatter(j + 1, sb1, ssem1)
            return 0
        lax.fori_loop(0, K // 2, pair, 0)
        wait_scatter(0, sb0, ssem0)
        wait_scatter(0, sb1, ssem1)

    plsc.subcore_barrier()
    for i in range(-(-NRBLK // NS)):
        @pl.when(i * NS + sid < NRBLK)
        def _():
            off = blk_off(i)
            pltpu.sync_copy(acc.at[pl.ds(off, RB)], zbuf)
            pltpu.sync_copy(zbuf, out_hbm.at[cid, pl.ds(off, RB)])


_layer = functools.partial(
    pl.kernel,
    out_type=jax.ShapeDtypeStruct((NC, N_NODES, HALF), jnp.float32),
    mesh=plsc.VectorSubcoreMesh(
        core_axis_name="c", subcore_axis_name="s",
        num_cores=NC, num_subcores=NS),
    scratch_types=[
        pltpu.VMEM_SHARED((N_NODES, HALF), jnp.float32),
        pltpu.VMEM((K, C), jnp.int32),
        pltpu.VMEM((K, C), jnp.int32),
        pltpu.VMEM((K, C), jnp.float32),
        pltpu.VMEM((C, HALF), jnp.float32),
        pltpu.VMEM((C, HALF), jnp.float32),
        pltpu.VMEM((C, HALF), jnp.float32),
        pltpu.VMEM((C, HALF), jnp.float32),
        pltpu.VMEM((RB, HALF), jnp.float32),
        pltpu.SemaphoreType.DMA,
        pltpu.SemaphoreType.DMA,
        pltpu.SemaphoreType.DMA,
        pltpu.SemaphoreType.DMA,
    ],
    compiler_params=pltpu.CompilerParams(use_tc_tiling_on_sc=False),
)(_layer_body)


def _mean_body(a, b, c, d, o):
    o[...] = (a[...] + b[...] + c[...] + d[...]) * 0.25


_FLAT = NC * N_NODES * HALF
_MROWS = _FLAT // 128
_MBLK = 1000

_mean4 = pl.pallas_call(
    _mean_body,
    grid=(_MROWS // _MBLK,),
    in_specs=[pl.BlockSpec((_MBLK, 128), lambda i: (i, 0))] * 4,
    out_specs=pl.BlockSpec((_MBLK, 128), lambda i: (i, 0)),
    out_shape=jax.ShapeDtypeStruct((_MROWS, 128), jnp.float32),
)


def kernel(adj_indices, adj_values, user_emb, item_emb):
    all_emb = jnp.concatenate([user_emb, item_emb], axis=0)
    x0 = jnp.stack([all_emb[:, :HALF], all_emb[:, HALF:]])
    npad = E_PAD - N_EDGES
    idx = adj_indices.astype(jnp.int32)
    row = jnp.concatenate([idx[0], jnp.zeros((npad,), jnp.int32)])
    col = jnp.concatenate([idx[1], jnp.zeros((npad,), jnp.int32)])
    val = jnp.concatenate([adj_values, jnp.zeros((npad,), jnp.float32)])
    row = row.reshape(E_PAD // C, C)
    col = col.reshape(E_PAD // C, C)
    val = val.reshape(E_PAD // C, C)

    x1 = _layer(col, row, val, x0)
    x2 = _layer(col, row, val, x1)
    x3 = _layer(col, row, val, x2)

    m = _mean4(x0.reshape(_MROWS, 128), x1.reshape(_MROWS, 128),
               x2.reshape(_MROWS, 128), x3.reshape(_MROWS, 128))
    m = m.reshape(NC, N_NODES, HALF)
    full = jnp.concatenate([m[0], m[1]], axis=1)
    return (full[:N_USERS], full[N_USERS:])


# speedup vs baseline: 1.3535x; 1.3535x over previous
import functools

import jax
import jax.numpy as jnp
from jax import lax
from jax.experimental import pallas as pl
from jax.experimental.pallas import tpu as pltpu
from jax.experimental.pallas import tpu_sc as plsc

N_USERS = 10000
N_ITEMS = 40000
N_NODES = N_USERS + N_ITEMS
EMB = 64
NC = 2
NQ = 2
QW = EMB // (NC * NQ)
N_EDGES = 800000
NS = 16
LANES = 16

C = 128
K = 4
E_PAD = 819200
CPT = E_PAD // NS // C
NSUP = CPT // K
RPT = N_NODES // NS
RZ = 125
NRZ = RPT // RZ


def _body(crv_hbm, x0_hbm, out_hbm,
          bufa, bufb, st0, st1, st2, st3,
          gb0, gb1, sb0, sb1, zbuf, m0, m1, m2, m3,
          gsem0, gsem1, ssem0, ssem1,
          stsem0, stsem1, stsem2, stsem3,
          msem0, msem1, msem2, msem3):
    cid = lax.axis_index("c")
    sid = lax.axis_index("s")
    r0 = sid * RPT
    tile_chunk0 = sid * CPT
    zvec = jnp.zeros((LANES,), jnp.float32)

    def fill_spmem(dst):
        def fz(z, _):
            pltpu.sync_copy(zbuf.at[pl.ds(0, RZ)], dst.at[pl.ds(r0 + z * RZ, RZ)])
            return 0
        lax.fori_loop(0, NRZ, fz, 0)

    stage = ((st0, stsem0), (st1, stsem1), (st2, stsem2), (st3, stsem3))

    def stage_issue(s, slot):
        st, sem = stage[slot]
        pltpu.async_copy(crv_hbm.at[pl.ds(tile_chunk0 + s * K, K)], st, sem)

    def stage_wait(s, slot):
        st, sem = stage[slot]
        pltpu.make_async_copy(
            crv_hbm.at[pl.ds(tile_chunk0 + s * K, K)], st, sem).wait()

    def edge_pass(src, dst):

        stage_issue(0, 0)
        stage_issue(1, 1)
        stage_issue(2, 2)
        stage_wait(0, 0)
        pltpu.async_copy(zbuf, dst.at[st0.at[0, 1]], ssem0, add=True)
        pltpu.async_copy(zbuf, dst.at[st0.at[0, 1]], ssem1, add=True)

        def chunk_loop(s, slot):
            st, _ = stage[slot]

            def issue(j, gb, sem):
                pltpu.async_copy(src.at[st.at[j, 0]], gb, sem)

            def wait_gather(j, gb, sem):
                pltpu.make_async_copy(src.at[st.at[j, 0]], gb, sem).wait()

            def scale(j, gb, sb):
                def group(g, _):
                    vvec = plsc.bitcast(
                        st[j, 2, pl.ds(LANES * g, LANES)], jnp.float32)
                    for ei in range(LANES):
                        e = LANES * g + ei
                        sb[e, pl.ds(0, LANES)] = gb[e, pl.ds(0, LANES)] * vvec[ei]
                    return 0
                lax.fori_loop(0, C // LANES, group, 0)

            def issue_scatter(j, sb, sem):
                pltpu.async_copy(sb, dst.at[st.at[j, 1]], sem, add=True)

            def wait_scatter(sb, sem):
                pltpu.make_async_copy(sb, dst.at[st.at[0, 1]], sem).wait()

            issue(0, gb0, gsem0)

            def pair(j2, _):
                j = 2 * j2
                issue(j + 1, gb1, gsem1)
                wait_gather(j, gb0, gsem0)
                wait_scatter(sb0, ssem0)
                scale(j, gb0, sb0)

                @pl.when(j2 < K // 2 - 1)
                def _():
                    issue(j + 2, gb0, gsem0)
                issue_scatter(j, sb0, ssem0)

                wait_gather(j + 1, gb1, gsem1)
                wait_scatter(sb1, ssem1)
                scale(j + 1, gb1, sb1)
                issue_scatter(j + 1, sb1, ssem1)
                return 0
            lax.fori_loop(0, K // 2, pair, 0)

        def sup4(s4, _):
            s = 4 * s4
            last = s4 >= NSUP // 4 - 1
            chunk_loop(s, 0)
            stage_wait(s + 1, 1)
            stage_issue(s + 3, 3)
            chunk_loop(s + 1, 1)
            stage_wait(s + 2, 2)

            @pl.when(jnp.logical_not(last))
            def _():
                stage_issue(s + 4, 0)
            chunk_loop(s + 2, 2)
            stage_wait(s + 3, 3)

            @pl.when(jnp.logical_not(last))
            def _():
                stage_issue(s + 5, 1)
            chunk_loop(s + 3, 3)

            @pl.when(jnp.logical_not(last))
            def _():
                stage_wait(s + 4, 0)
                stage_issue(s + 6, 2)
            return 0
        lax.fori_loop(0, NSUP // 4, sup4, 0)

        pltpu.make_async_copy(sb0, dst.at[st0.at[0, 1]], ssem0).wait()
        pltpu.make_async_copy(sb1, dst.at[st0.at[0, 1]], ssem1).wait()

    def zz(i, _):
        zbuf[i, pl.ds(0, LANES)] = zvec
        return 0
    lax.fori_loop(0, C, zz, 0)

    def quarter(q, _):
        def sx(z, _):
            blk = pl.ds(r0 + z * RZ, RZ)
            pltpu.sync_copy(x0_hbm.at[cid, q].at[blk], m0)
            pltpu.sync_copy(m0, bufa.at[blk])
            return 0
        lax.fori_loop(0, NRZ, sx, 0)
        fill_spmem(bufb)
        plsc.subcore_barrier()

        def lbody(l, _):
            even = (l == 0) | (l == 2)

            @pl.when(even)
            def _():
                edge_pass(bufa, bufb)

            @pl.when(jnp.logical_not(even))
            def _():
                edge_pass(bufb, bufa)
            plsc.subcore_barrier()

            @pl.when(l == 0)
            def _():
                def px(z, _):
                    blk = pl.ds(r0 + z * RZ, RZ)
                    xblk = pl.ds(N_NODES + r0 + z * RZ, RZ)
                    pltpu.sync_copy(bufb.at[blk], m0)
                    pltpu.sync_copy(m0, out_hbm.at[cid, q].at[xblk])
                    return 0
                lax.fori_loop(0, NRZ, px, 0)
                fill_spmem(bufa)

            @pl.when(l == 1)
            def _():
                fill_spmem(bufb)

            @pl.when(l < 2)
            def _():
                plsc.subcore_barrier()
            return 0
        lax.fori_loop(0, 3, lbody, 0)

        def mz(z, _):
            blk = pl.ds(r0 + z * RZ, RZ)
            xblk = pl.ds(N_NODES + r0 + z * RZ, RZ)
            pltpu.async_copy(x0_hbm.at[cid, q].at[blk], m0, msem0)
            pltpu.async_copy(out_hbm.at[cid, q].at[xblk], m1, msem1)
            pltpu.async_copy(bufa.at[blk], m2, msem2)
            pltpu.async_copy(bufb.at[blk], m3, msem3)
            pltpu.make_async_copy(x0_hbm.at[cid, q].at[blk], m0, msem0).wait()
            pltpu.make_async_copy(out_hbm.at[cid, q].at[xblk], m1, msem1).wait()
            pltpu.make_async_copy(bufa.at[blk], m2, msem2).wait()
            pltpu.make_async_copy(bufb.at[blk], m3, msem3).wait()

            def mrow(i, _):
                s16 = pl.ds(0, LANES)
                m0[i, s16] = (m0[i, s16] + m1[i, s16]
                              + m2[i, s16] + m3[i, s16]) * 0.25
                return 0
            lax.fori_loop(0, RZ, mrow, 0)
            pltpu.sync_copy(m0, out_hbm.at[cid, q].at[blk])
            return 0
        lax.fori_loop(0, NRZ, mz, 0)
        plsc.subcore_barrier()
        return 0

    lax.fori_loop(0, NQ, quarter, 0)


_lightgcn = functools.partial(
    pl.kernel,
    out_type=jax.ShapeDtypeStruct((NC, NQ, 2 * N_NODES, QW), jnp.float32),
    mesh=plsc.VectorSubcoreMesh(
        core_axis_name="c", subcore_axis_name="s",
        num_cores=NC, num_subcores=NS),
    scratch_types=[
        pltpu.VMEM_SHARED((N_NODES, QW), jnp.float32),
        pltpu.VMEM_SHARED((N_NODES, QW), jnp.float32),
        pltpu.VMEM((K, 3, C), jnp.int32),
        pltpu.VMEM((K, 3, C), jnp.int32),
        pltpu.VMEM((K, 3, C), jnp.int32),
        pltpu.VMEM((K, 3, C), jnp.int32),
        pltpu.VMEM((C, QW), jnp.float32),
        pltpu.VMEM((C, QW), jnp.float32),
        pltpu.VMEM((C, QW), jnp.float32),
        pltpu.VMEM((C, QW), jnp.float32),
        pltpu.VMEM((C, QW), jnp.float32),
        pltpu.VMEM((RZ, QW), jnp.float32),
        pltpu.VMEM((RZ, QW), jnp.float32),
        pltpu.VMEM((RZ, QW), jnp.float32),
        pltpu.VMEM((RZ, QW), jnp.float32),
    ] + [pltpu.SemaphoreType.DMA] * 12,
    compiler_params=pltpu.CompilerParams(
        use_tc_tiling_on_sc=False, needs_layout_passes=False),
)(_body)


def kernel(adj_indices, adj_values, user_emb, item_emb):
    all_emb = jnp.concatenate([user_emb, item_emb], axis=0)
    x0 = all_emb.reshape(N_NODES, NC * NQ, QW).transpose(1, 0, 2)
    x0 = x0.reshape(NC, NQ, N_NODES, QW)
    npad = E_PAD - N_EDGES
    idx = adj_indices.astype(jnp.int32)
    row = jnp.concatenate([idx[0], jnp.zeros((npad,), jnp.int32)])
    col = jnp.concatenate([idx[1], jnp.zeros((npad,), jnp.int32)])
    val = jnp.concatenate([adj_values, jnp.zeros((npad,), jnp.float32)])
    vali = jax.lax.bitcast_convert_type(val, jnp.int32)
    crv = jnp.stack([col.reshape(E_PAD // C, C),
                     row.reshape(E_PAD // C, C),
                     vali.reshape(E_PAD // C, C)], axis=1)

    out = _lightgcn(crv, x0)
    m = out[:, :, :N_NODES]
    full = m.transpose(2, 0, 1, 3).reshape(N_NODES, EMB)
    return (full[:N_USERS], full[N_USERS:])
